# baseline (speedup 1.0000x reference)
import functools

import jax
import jax.numpy as jnp
from jax import lax
from jax.experimental import pallas as pl
from jax.experimental.pallas import tpu as pltpu
from jax.experimental.pallas import tpu_sc as plsc

_N_EMBD = 768
_BATCH = 32
_SEQ = 1024
_NC, _NS = 2, 16
_NW = _NC * _NS
_K = _SEQ // _NW
_L = 16


def _emb_body(data_hbm, wte_hbm, wpe_hbm, out_hbm, idx_v, pos_v, buf_v, sem):
    wid = lax.axis_index("s") * _NC + lax.axis_index("c")
    base_s = wid * _K

    pltpu.sync_copy(wpe_hbm.at[pl.ds(base_s, _K)], pos_v)

    def per_batch(b, _):
        pltpu.sync_copy(data_hbm.at[pl.ds(b * _SEQ + base_s, _K)], idx_v)
        pltpu.async_copy(wte_hbm.at[idx_v], buf_v, sem).wait()

        def per_row(r, _):
            for j in range(_N_EMBD // _L):
                sl = pl.ds(j * _L, _L)
                plsc.addupdate(buf_v.at[r, sl], pos_v[r, sl])
            return 0

        lax.fori_loop(0, _K, per_row, 0)
        pltpu.sync_copy(buf_v, out_hbm.at[b, pl.ds(base_s, _K)])
        return 0

    lax.fori_loop(0, _BATCH, per_batch, 0)


@jax.jit
def kernel(data, wte, wpe):
    mesh = plsc.VectorSubcoreMesh(
        core_axis_name="c", subcore_axis_name="s",
        num_cores=_NC, num_subcores=_NS,
    )
    run = functools.partial(
        pl.kernel,
        out_type=jax.ShapeDtypeStruct((_BATCH, _SEQ, _N_EMBD), jnp.float32),
        mesh=mesh,
        scratch_types=[
            pltpu.VMEM((_K,), jnp.int32),
            pltpu.VMEM((_K, _N_EMBD), jnp.float32),
            pltpu.VMEM((_K, _N_EMBD), jnp.float32),
            pltpu.SemaphoreType.DMA,
        ],
    )(_emb_body)
    return run(data.reshape(-1), wte, wpe)


# speedup vs baseline: 1.8243x; 1.8243x over previous
import functools

import jax
import jax.numpy as jnp
from jax import lax
from jax.experimental import pallas as pl
from jax.experimental.pallas import tpu as pltpu
from jax.experimental.pallas import tpu_sc as plsc

_N_EMBD = 768
_BATCH = 32
_SEQ = 1024
_NC, _NS = 2, 16
_NW = _NC * _NS
_K = _SEQ // _NW
_L = 16
_NBUF = 4


def _emb_body(data_hbm, wte_hbm, wpe_hbm, out_hbm,
              idx_all, pos_v, bufs, isem, gsems, osems):
    wid = lax.axis_index("s") * _NC + lax.axis_index("c")
    base_s = wid * _K

    pltpu.sync_copy(wpe_hbm.at[pl.ds(base_s, _K)], pos_v)

    for b in range(_BATCH):
        pltpu.async_copy(
            data_hbm.at[pl.ds(b * _SEQ + base_s, _K)], idx_all.at[b], isem)
    for b in range(_BATCH):
        pltpu.make_async_copy(
            data_hbm.at[pl.ds(b * _SEQ + base_s, _K)], idx_all.at[b], isem
        ).wait()

    def gather(b, p):
        pltpu.async_copy(wte_hbm.at[idx_all.at[b]], bufs[p], gsems[p])

    def gather_wait(b, p):
        pltpu.make_async_copy(
            wte_hbm.at[idx_all.at[b]], bufs[p], gsems[p]).wait()

    def out_wait(p):
        pltpu.make_async_copy(wte_hbm.at[pl.ds(0, _K)], bufs[p], osems[p]).wait()

    gather(0, 0)
    gather(1, 1)

    def step(i, _):
        for p in range(_NBUF):
            b = i * _NBUF + p
            q = (p + 2) % _NBUF

            @pl.when(b + 2 < _BATCH)
            def _():
                @pl.when(b >= 2)
                def _():
                    out_wait(q)
                gather(b + 2, q)

            gather_wait(b, p)
            buf = bufs[p]

            def per_row(r, _):
                for j in range(_N_EMBD // _L):
                    sl = pl.ds(j * _L, _L)
                    plsc.addupdate(buf.at[r, sl], pos_v[r, sl])
                return 0

            lax.fori_loop(0, _K, per_row, 0)
            pltpu.async_copy(buf, out_hbm.at[b, pl.ds(base_s, _K)], osems[p])
        return 0

    lax.fori_loop(0, _BATCH // _NBUF, step, 0)

    for p in range(_NBUF):
        out_wait(p)


@jax.jit
def kernel(data, wte, wpe):
    mesh = plsc.VectorSubcoreMesh(
        core_axis_name="c", subcore_axis_name="s",
        num_cores=_NC, num_subcores=_NS,
    )
    run = functools.partial(
        pl.kernel,
        out_type=jax.ShapeDtypeStruct((_BATCH, _SEQ, _N_EMBD), jnp.float32),
        mesh=mesh,
        scratch_types=[
            pltpu.VMEM((_BATCH, _K), jnp.int32),
            pltpu.VMEM((_K, _N_EMBD), jnp.float32),
            tuple(pltpu.VMEM((_K, _N_EMBD), jnp.float32)
                  for _ in range(_NBUF)),
            pltpu.SemaphoreType.DMA,
            tuple(pltpu.SemaphoreType.DMA for _ in range(_NBUF)),
            tuple(pltpu.SemaphoreType.DMA for _ in range(_NBUF)),
        ],
    )(_emb_body)
    return run(data.reshape(-1), wte, wpe)
